# baseline (speedup 1.0000x reference)
import functools

import jax
import jax.numpy as jnp
from jax import lax
from jax.experimental import pallas as pl
from jax.experimental.pallas import tpu as pltpu
from jax.experimental.pallas import tpu_sc as plsc

N = 8192
H = 128
F_IN = 128
HEADS = 3
E = 262144

_NTILES = 32
_CHUNK = 128
_RPS = N // 16

_SC_MESH = plsc.VectorSubcoreMesh(core_axis_name="c", subcore_axis_name="s")

_TPC = 16
_NCHT = 144
_NE = _TPC * _CHUNK * _NCHT


def _wsegsum_factory():
    ncht = _NE // (_NTILES * _CHUNK)

    dnums = lax.GatherDimensionNumbers(
        offset_dims=(), collapsed_slice_dims=(0,), start_index_map=(0,))

    def _bcast(w16, i):
        idx = jnp.full((16, 1), i, jnp.int32)
        return lax.gather(w16, idx, dnums, (1,),
                          mode=lax.GatherScatterMode.PROMISE_IN_BOUNDS)

    def _mul_rows(rows, wts_v, j):
        for g in range(_CHUNK // 16):
            w16 = wts_v[j, pl.ds(g * 16, 16)]
            for i in range(16):
                wv = _bcast(w16, i)
                r = g * 16 + i
                for cb in range(H // 16):
                    sl = pl.ds(cb * 16, 16)
                    rows[r, sl] = rows[r, sl] * wv

    def body(h_hbm, src_hbm, dst_hbm, wts_hbm, out_hbm,
             src_v, dst_v, wts_v, rows_a, rows_b, acc, sem_a, sem_b):
        c = lax.axis_index("c")
        s = lax.axis_index("s")
        tid = s * 2 + c
        cbase = tid * ncht

        pltpu.sync_copy(src_hbm.at[pl.ds(cbase, ncht)], src_v)
        pltpu.sync_copy(dst_hbm.at[pl.ds(cbase, ncht)], dst_v)
        pltpu.sync_copy(wts_hbm.at[pl.ds(cbase, ncht)], wts_v)

        def zrow(i, carry):
            for j in range(H // 16):
                rows_b[i, pl.ds(j * 16, 16)] = jnp.zeros((16,), jnp.float32)
            return carry
        lax.fori_loop(0, _CHUNK, zrow, 0)
        for k in range(_RPS // _CHUNK):
            pltpu.sync_copy(rows_b,
                            acc.at[pl.ds(s * _RPS + k * _CHUNK, _CHUNK)])
        plsc.subcore_barrier()

        pltpu.async_copy(h_hbm.at[src_v.at[0]], rows_a, sem_a)

        def step(k, carry):
            j0 = 2 * k
            j1 = 2 * k + 1
            pltpu.make_async_copy(h_hbm.at[src_v.at[0]], rows_a, sem_a).wait()
            pltpu.async_copy(h_hbm.at[src_v.at[j1]], rows_b, sem_b)
            _mul_rows(rows_a, wts_v, j0)
            pltpu.sync_copy(rows_a, acc.at[dst_v.at[j0]], add=True)
            pltpu.make_async_copy(h_hbm.at[src_v.at[0]], rows_b, sem_b).wait()
            jn = jnp.minimum(j0 + 2, ncht - 1)
            pltpu.async_copy(h_hbm.at[src_v.at[jn]], rows_a, sem_a)
            _mul_rows(rows_b, wts_v, j1)
            pltpu.sync_copy(rows_b, acc.at[dst_v.at[j1]], add=True)
            return carry

        lax.fori_loop(0, ncht // 2, step, 0)
        pltpu.make_async_copy(h_hbm.at[src_v.at[0]], rows_a, sem_a).wait()

        plsc.subcore_barrier()
        for k in range(_RPS // _CHUNK):
            r0 = s * _RPS + k * _CHUNK
            pltpu.sync_copy(acc.at[pl.ds(r0, _CHUNK)],
                            out_hbm.at[c, pl.ds(r0, _CHUNK)])

    return pl.kernel(
        body,
        out_type=jax.ShapeDtypeStruct((2, N, H), jnp.float32),
        mesh=_SC_MESH,
        scratch_types=[
            pltpu.VMEM((ncht, _CHUNK), jnp.int32),
            pltpu.VMEM((ncht, _CHUNK), jnp.int32),
            pltpu.VMEM((ncht, _CHUNK), jnp.float32),
            pltpu.VMEM((_CHUNK, H), jnp.float32),
            pltpu.VMEM((_CHUNK, H), jnp.float32),
            pltpu.VMEM_SHARED((N, H), jnp.float32),
            pltpu.SemaphoreType.DMA,
            pltpu.SemaphoreType.DMA,
        ],
    )


_WSEGSUM = _wsegsum_factory()


def _edge_softmax_factory():
    ncht = _NE // (_NTILES * _CHUNK)

    def body(s_hbm, d_hbm, asad_hbm, ex_hbm, ss_hbm,
             src_v, dst_v, tb, hist, exb, sem):
        c = lax.axis_index("c")
        s = lax.axis_index("s")
        tid = s * 2 + c
        cbase = tid * ncht

        pltpu.sync_copy(s_hbm.at[pl.ds(cbase, ncht)], src_v)
        pltpu.sync_copy(d_hbm.at[pl.ds(cbase, ncht)], dst_v)
        pltpu.sync_copy(asad_hbm, tb)

        def zh(i, carry):
            z = jnp.zeros((16,), jnp.float32)
            for k in range(3):
                hist[k, pl.ds(i * 16, 16)] = z
            return carry
        lax.fori_loop(0, N // 16, zh, 0)

        iota = lax.iota(jnp.int32, 16)
        nreal = jnp.int32(E + N)

        def outer(ko, carry):
            for kc in range(8):
                j = ko * 8 + kc
                for g in range(8):
                    sl16 = pl.ds(g * 16, 16)
                    s16 = src_v[j, sl16]
                    d16 = dst_v[j, sl16]
                    gbase = (cbase + j) * _CHUNK + g * 16
                    msk = (gbase + iota) < nreal
                    for k in range(3):
                        av = plsc.load_gather(tb.at[k], [s16])
                        bv = plsc.load_gather(tb.at[3 + k], [d16])
                        ev = av + bv
                        ev = jnp.where(ev > 0, ev, 0.2 * ev)
                        xv = jnp.where(msk, jnp.exp(ev), 0.0)
                        plsc.addupdate_scatter(hist.at[k], [d16], xv)
                        exb[k, kc, sl16] = xv
            for k in range(3):
                pltpu.sync_copy(
                    exb.at[k],
                    ex_hbm.at[k, pl.ds(cbase + ko * 8, 8)])
            return carry

        lax.fori_loop(0, ncht // 8, outer, 0)

        pltpu.sync_copy(hist, ss_hbm.at[tid])

    return pl.kernel(
        body,
        out_type=(
            jax.ShapeDtypeStruct((3, _NE // _CHUNK, _CHUNK), jnp.float32),
            jax.ShapeDtypeStruct((_NTILES, 3, N), jnp.float32),
        ),
        mesh=_SC_MESH,
        scratch_types=[
            pltpu.VMEM((ncht, _CHUNK), jnp.int32),
            pltpu.VMEM((ncht, _CHUNK), jnp.int32),
            pltpu.VMEM((6, N), jnp.float32),
            pltpu.VMEM((3, N), jnp.float32),
            pltpu.VMEM((3, 8, _CHUNK), jnp.float32),
            pltpu.SemaphoreType.DMA,
        ],
        compiler_params=pltpu.CompilerParams(
            use_tc_tiling_on_sc=False, needs_layout_passes=False),
    )


def _deg_factory():
    ncht = _NE // (_NTILES * _CHUNK)

    def body(s_hbm, deg_hbm, src_v, hist, sem):
        c = lax.axis_index("c")
        s = lax.axis_index("s")
        tid = s * 2 + c
        cbase = tid * ncht

        pltpu.sync_copy(s_hbm.at[pl.ds(cbase, ncht)], src_v)

        def zh(i, carry):
            hist[pl.ds(i * 16, 16)] = jnp.zeros((16,), jnp.float32)
            return carry
        lax.fori_loop(0, N // 16, zh, 0)

        iota = lax.iota(jnp.int32, 16)
        nreal = jnp.int32(E)

        def outer(j, carry):
            for g in range(8):
                sl16 = pl.ds(g * 16, 16)
                s16 = src_v[j, sl16]
                gbase = (cbase + j) * _CHUNK + g * 16
                ones = jnp.where((gbase + iota) < nreal, 1.0, 0.0)
                plsc.addupdate_scatter(hist, [s16], ones)
            return carry

        lax.fori_loop(0, ncht, outer, 0)
        pltpu.sync_copy(hist, deg_hbm.at[tid])

    return pl.kernel(
        body,
        out_type=jax.ShapeDtypeStruct((_NTILES, N), jnp.float32),
        mesh=_SC_MESH,
        scratch_types=[
            pltpu.VMEM((ncht, _CHUNK), jnp.int32),
            pltpu.VMEM((N,), jnp.float32),
            pltpu.SemaphoreType.DMA,
        ],
        compiler_params=pltpu.CompilerParams(
            use_tc_tiling_on_sc=False, needs_layout_passes=False),
    )


_EDGE_SOFTMAX = _edge_softmax_factory()
_DEG = _deg_factory()


_BLK = 1024


def _mm_relu_kernel(x_ref, w_ref, b_ref, o_ref):
    o_ref[...] = jnp.maximum(
        jnp.dot(x_ref[...], w_ref[...], preferred_element_type=jnp.float32,
                  precision=lax.Precision.HIGHEST)
        + b_ref[...], 0.0)


def _emb(x, W, b):
    return pl.pallas_call(
        _mm_relu_kernel,
        grid=(N // _BLK,),
        in_specs=[
            pl.BlockSpec((_BLK, F_IN), lambda i: (i, 0)),
            pl.BlockSpec((F_IN, H), lambda i: (0, 0)),
            pl.BlockSpec((1, H), lambda i: (0, 0)),
        ],
        out_specs=pl.BlockSpec((_BLK, H), lambda i: (i, 0)),
        out_shape=jax.ShapeDtypeStruct((N, H), jnp.float32),
    )(x, W, b.reshape(1, H))


def _mm3_kernel(x0_ref, x1_ref, x2_ref, w_ref, b_ref, o_ref):
    acc = jnp.dot(x0_ref[...], w_ref[0], preferred_element_type=jnp.float32,
                  precision=lax.Precision.HIGHEST)
    acc += jnp.dot(x1_ref[...], w_ref[1], preferred_element_type=jnp.float32,
                  precision=lax.Precision.HIGHEST)
    acc += jnp.dot(x2_ref[...], w_ref[2], preferred_element_type=jnp.float32,
                  precision=lax.Precision.HIGHEST)
    o_ref[...] = acc + b_ref[...]


def _mm3(x0, x1, x2, W3, b):
    return pl.pallas_call(
        _mm3_kernel,
        grid=(N // _BLK,),
        in_specs=[
            pl.BlockSpec((_BLK, H), lambda i: (i, 0)),
            pl.BlockSpec((_BLK, H), lambda i: (i, 0)),
            pl.BlockSpec((_BLK, H), lambda i: (i, 0)),
            pl.BlockSpec((3, H, H), lambda i: (0, 0, 0)),
            pl.BlockSpec((1, H), lambda i: (0, 0)),
        ],
        out_specs=pl.BlockSpec((_BLK, H), lambda i: (i, 0)),
        out_shape=jax.ShapeDtypeStruct((N, H), jnp.float32),
    )(x0, x1, x2, W3, b.reshape(1, H))


def _stats_kernel(x_ref, cb_ref, o_ref):
    i = pl.program_id(0)
    x = x_ref[...] + cb_ref[...]
    s1 = jnp.sum(x, axis=0, keepdims=True)
    s2 = jnp.sum(x * x, axis=0, keepdims=True)
    st = jnp.concatenate([s1, s2], axis=0)

    @pl.when(i == 0)
    def _():
        o_ref[...] = st

    @pl.when(i > 0)
    def _():
        o_ref[...] = o_ref[...] + st


def _stats(x, cb):
    w = x.shape[1]
    return pl.pallas_call(
        _stats_kernel,
        grid=(N // _BLK,),
        in_specs=[
            pl.BlockSpec((_BLK, w), lambda i: (i, 0)),
            pl.BlockSpec((1, w), lambda i: (0, 0)),
        ],
        out_specs=pl.BlockSpec((2, w), lambda i: (0, 0)),
        out_shape=jax.ShapeDtypeStruct((2, w), jnp.float32),
    )(x, cb.reshape(1, w))


def _gatw_kernel(x_ref, c1_ref, c0_ref, w_ref, att_ref, hw_ref, as_ref):
    xb = x_ref[...] * c1_ref[...] + c0_ref[...]
    hw = jnp.dot(xb, w_ref[...], preferred_element_type=jnp.float32,
                  precision=lax.Precision.HIGHEST)
    hw_ref[...] = hw
    as_ref[...] = jnp.dot(hw, att_ref[...], preferred_element_type=jnp.float32,
                  precision=lax.Precision.HIGHEST)


def _gatw(x, c1, c0, W, attT):
    return pl.pallas_call(
        _gatw_kernel,
        grid=(N // _BLK,),
        in_specs=[
            pl.BlockSpec((_BLK, H), lambda i: (i, 0)),
            pl.BlockSpec((1, H), lambda i: (0, 0)),
            pl.BlockSpec((1, H), lambda i: (0, 0)),
            pl.BlockSpec((H, HEADS * H), lambda i: (0, 0)),
            pl.BlockSpec((HEADS * H, 8), lambda i: (0, 0)),
        ],
        out_specs=[
            pl.BlockSpec((_BLK, HEADS * H), lambda i: (i, 0)),
            pl.BlockSpec((_BLK, 8), lambda i: (i, 0)),
        ],
        out_shape=[
            jax.ShapeDtypeStruct((N, HEADS * H), jnp.float32),
            jax.ShapeDtypeStruct((N, 8), jnp.float32),
        ],
    )(x, c1.reshape(1, H), c0.reshape(1, H), W, attT)


def _linaff_kernel(x_ref, c1_ref, c0_ref, w_ref, b_ref, o_ref):
    xb = x_ref[...] * c1_ref[...] + c0_ref[...]
    o_ref[...] = (jnp.dot(xb, w_ref[...], preferred_element_type=jnp.float32,
                  precision=lax.Precision.HIGHEST)
                  + b_ref[...])


def _linaff(x, c1, c0, W, b):
    kin = x.shape[1]
    return pl.pallas_call(
        _linaff_kernel,
        grid=(N // _BLK,),
        in_specs=[
            pl.BlockSpec((_BLK, kin), lambda i: (i, 0)),
            pl.BlockSpec((1, kin), lambda i: (0, 0)),
            pl.BlockSpec((1, kin), lambda i: (0, 0)),
            pl.BlockSpec((kin, H), lambda i: (0, 0)),
            pl.BlockSpec((1, H), lambda i: (0, 0)),
        ],
        out_specs=pl.BlockSpec((_BLK, H), lambda i: (i, 0)),
        out_shape=jax.ShapeDtypeStruct((N, H), jnp.float32),
    )(x, c1.reshape(1, kin), c0.reshape(1, kin), W, b.reshape(1, H))


def _affine_relu_kernel(x_ref, c1_ref, c0_ref, o_ref):
    o_ref[...] = jnp.maximum(x_ref[...] * c1_ref[...] + c0_ref[...], 0.0)


def _affine_relu(x, c1, c0):
    return pl.pallas_call(
        _affine_relu_kernel,
        grid=(N // _BLK,),
        in_specs=[
            pl.BlockSpec((_BLK, H), lambda i: (i, 0)),
            pl.BlockSpec((1, H), lambda i: (0, 0)),
            pl.BlockSpec((1, H), lambda i: (0, 0)),
        ],
        out_specs=pl.BlockSpec((_BLK, H), lambda i: (i, 0)),
        out_shape=jax.ShapeDtypeStruct((N, H), jnp.float32),
    )(x, c1.reshape(1, H), c0.reshape(1, H))


def _rowscale_kernel(x_ref, r_ref, o_ref):
    o_ref[...] = x_ref[...] * r_ref[...]


def _rowscale(x, r):
    return pl.pallas_call(
        _rowscale_kernel,
        grid=(N // _BLK,),
        in_specs=[
            pl.BlockSpec((_BLK, H), lambda i: (i, 0)),
            pl.BlockSpec((_BLK, 1), lambda i: (i, 0)),
        ],
        out_specs=pl.BlockSpec((_BLK, H), lambda i: (i, 0)),
        out_shape=jax.ShapeDtypeStruct((N, H), jnp.float32),
    )(x, r.reshape(N, 1))


def _add2rs_kernel(p_ref, r_ref, o_ref):
    o_ref[...] = (p_ref[0] + p_ref[1]) * r_ref[...]


def _add2rs(p, r):
    return pl.pallas_call(
        _add2rs_kernel,
        grid=(N // _BLK,),
        in_specs=[
            pl.BlockSpec((2, _BLK, H), lambda i: (0, i, 0)),
            pl.BlockSpec((_BLK, 1), lambda i: (i, 0)),
        ],
        out_specs=pl.BlockSpec((_BLK, H), lambda i: (i, 0)),
        out_shape=jax.ShapeDtypeStruct((N, H), jnp.float32),
    )(p, r.reshape(N, 1))


def _add4_kernel(x_ref, c1_ref, c0_ref, a_ref, b_ref, d_ref, o_ref):
    o_ref[...] = (x_ref[...] * c1_ref[...] + c0_ref[...]
                  + a_ref[...] + b_ref[...] + d_ref[...])


def _add4(x, c1, c0, g0, g1, g2):
    return pl.pallas_call(
        _add4_kernel,
        grid=(N // _BLK,),
        in_specs=[
            pl.BlockSpec((_BLK, H), lambda i: (i, 0)),
            pl.BlockSpec((1, H), lambda i: (0, 0)),
            pl.BlockSpec((1, H), lambda i: (0, 0)),
            pl.BlockSpec((_BLK, H), lambda i: (i, 0)),
            pl.BlockSpec((_BLK, H), lambda i: (i, 0)),
            pl.BlockSpec((_BLK, H), lambda i: (i, 0)),
        ],
        out_specs=pl.BlockSpec((_BLK, H), lambda i: (i, 0)),
        out_shape=jax.ShapeDtypeStruct((N, H), jnp.float32),
    )(x, c1.reshape(1, H), c0.reshape(1, H), g0, g1, g2)


def _reduce32_kernel_rsqrt(x_ref, o_ref):
    s = jnp.sum(x_ref[...], axis=0, keepdims=True)
    o_ref[...] = jnp.where(s > 0, lax.rsqrt(jnp.maximum(s, 1e-30)), 0.0)


def _reduce32_kernel_recip(x_ref, o_ref):
    s = jnp.sum(x_ref[...], axis=0, keepdims=True)
    o_ref[...] = 1.0 / s


def _reduce32(x, post):
    x = x.reshape(_NTILES, -1)
    mm = x.shape[1]
    blk = 2048
    kern = (_reduce32_kernel_rsqrt if post == 'rsqrt'
            else _reduce32_kernel_recip)
    out = pl.pallas_call(
        kern,
        grid=(mm // blk,),
        in_specs=[pl.BlockSpec((_NTILES, blk), lambda i: (0, i))],
        out_specs=pl.BlockSpec((1, blk), lambda i: (0, i)),
        out_shape=jax.ShapeDtypeStruct((1, mm), jnp.float32),
    )(x)
    return out.reshape(mm)


def _bn_affine(x, cb, gamma, beta):
    mu = _stats(x, cb)[0] / N
    var = _stats(x, cb - mu)[1] / N
    c1 = gamma / jnp.sqrt(var + 1e-5)
    c0 = beta - mu * c1
    return c1, c0


def kernel(x, params, edge_index):
    src = edge_index[0]
    dst = edge_index[1]
    loop = jnp.arange(N, dtype=src.dtype)
    zc = jnp.zeros((_NE - E,), src.dtype)
    zg = jnp.zeros((_NE - (E + N),), src.dtype)
    src2 = jnp.concatenate([src, zc]).reshape(_NE // _CHUNK, _CHUNK)
    dst2 = jnp.concatenate([dst, zc]).reshape(_NE // _CHUNK, _CHUNK)
    s2 = jnp.concatenate([src, loop, zg]).reshape(_NE // _CHUNK, _CHUNK)
    d2 = jnp.concatenate([dst, loop, zg]).reshape(_NE // _CHUNK, _CHUNK)
    wts_cheb = jnp.concatenate(
        [jnp.ones((E,), jnp.float32), jnp.zeros((_NE - E,), jnp.float32)]
    ).reshape(src2.shape)

    dinv = _reduce32(_DEG(src2), 'rsqrt')
    ndinv = -dinv
    p = params
    zH = jnp.zeros((H,), jnp.float32)

    def prop(h):
        hs = _rowscale(h, dinv)
        return _add2rs(_WSEGSUM(hs, src2, dst2, wts_cheb), ndinv)

    cur = _emb(x, p['W_emb'], p['b_emb'])
    gats = []
    for l in range(3):
        W3 = jnp.stack([p[f'cheb_W{l}'][0] - p[f'cheb_W{l}'][2],
                        p[f'cheb_W{l}'][1],
                        2.0 * p[f'cheb_W{l}'][2]])
        tx1 = prop(cur)
        p2 = prop(tx1)
        cheb_out = _mm3(cur, tx1, p2, W3, p[f'cheb_b{l}'])
        c1, c0 = _bn_affine(cheb_out, zH, p['bn_gamma'], p['bn_beta'])
        attT = jnp.zeros((HEADS * H, 8), jnp.float32)
        for k in range(HEADS):
            attT = attT.at[k * H:(k + 1) * H, k].set(p[f'gat{l}_att_src'][k])
            attT = attT.at[k * H:(k + 1) * H, 3 + k].set(
                p[f'gat{l}_att_dst'][k])
        hw, as8 = _gatw(cheb_out, c1, c0, p[f'gat{l}_W'], attT)
        asad = as8[:, :6].T
        ex, ssp = _EDGE_SOFTMAX(s2, d2, asad)
        rinv = _reduce32(ssp, 'recip').reshape(HEADS, N)
        aggs = [_add2rs(_WSEGSUM(hw[:, k * H:(k + 1) * H], s2, d2, ex[k]),
                        rinv[k]) for k in range(HEADS)]
        gat_cat = jnp.concatenate(aggs, axis=1)
        gc1, gc0 = _bn_affine(gat_cat, p[f'gat{l}_bias'],
                              p[f'bn{l}_gamma'], p[f'bn{l}_beta'])
        gc0 = gc0 + p[f'gat{l}_bias'] * gc1
        gats.append(_linaff(gat_cat, gc1, gc0,
                            p[f'lin{l}_W'], p[f'lin{l}_b']))
        if l < 2:
            cur = _affine_relu(cheb_out, c1, c0)
    return _add4(cheb_out, c1, c0, gats[0], gats[1], gats[2])


# speedup vs baseline: 1.2586x; 1.2586x over previous
import functools

import jax
import jax.numpy as jnp
from jax import lax
from jax.experimental import pallas as pl
from jax.experimental.pallas import tpu as pltpu
from jax.experimental.pallas import tpu_sc as plsc

N = 8192
H = 128
F_IN = 128
HEADS = 3
E = 262144

_NTILES = 32
_CHUNK = 128
_RPS = N // 16

_SC_MESH = plsc.VectorSubcoreMesh(core_axis_name="c", subcore_axis_name="s")


_HC = H // 2
_TPC = 16
_NCHT = 144
_NE = _TPC * _CHUNK * _NCHT


def _wsegsum_factory():
    ncht = _NCHT

    dnums = lax.GatherDimensionNumbers(
        offset_dims=(), collapsed_slice_dims=(0,), start_index_map=(0,))

    def _bcast(w16, i):
        idx = jnp.full((16, 1), i, jnp.int32)
        return lax.gather(w16, idx, dnums, (1,),
                          mode=lax.GatherScatterMode.PROMISE_IN_BOUNDS)

    def _mul_rows(rows, src_v, dst_v, wts_v, ta_v, tb_v, j):
        for g in range(_CHUNK // 16):
            sl16 = pl.ds(g * 16, 16)
            w16 = wts_v[j, sl16]
            ga = plsc.load_gather(ta_v, [dst_v[j, sl16]])
            gb = plsc.load_gather(tb_v, [src_v[j, sl16]])
            w16 = w16 * ga * gb
            for i in range(16):
                wv = _bcast(w16, i)
                r = g * 16 + i
                for cb in range(_HC // 16):
                    sl = pl.ds(cb * 16, 16)
                    rows[r, sl] = rows[r, sl] * wv

    def body(h_hbm, src_hbm, dst_hbm, wts_hbm, ta_hbm, tb_hbm, out_hbm,
             src_v, dst_v, wts_v, ta_v, tb_v, rows_a, acc, sem_a):
        c = lax.axis_index("c")
        s = lax.axis_index("s")
        cbase = s * ncht
        off = c * N

        pltpu.sync_copy(ta_hbm, ta_v)
        pltpu.sync_copy(tb_hbm, tb_v)

        def zrow(i, carry):
            for j in range(_HC // 16):
                rows_a[i, pl.ds(j * 16, 16)] = jnp.zeros((16,), jnp.float32)
            return carry
        lax.fori_loop(0, _CHUNK, zrow, 0)
        for k in range(_RPS // _CHUNK):
            pltpu.sync_copy(rows_a,
                            acc.at[pl.ds(s * _RPS + k * _CHUNK, _CHUNK)])
        plsc.subcore_barrier()

        def outer(ko, carry):
            base = cbase + ko * 16
            pltpu.sync_copy(src_hbm.at[pl.ds(base, 16)], src_v)
            pltpu.sync_copy(dst_hbm.at[pl.ds(base, 16)], dst_v)
            pltpu.sync_copy(wts_hbm.at[pl.ds(base, 16)], wts_v)

            def offr(i, cr):
                for g in range(_CHUNK // 16):
                    sl = pl.ds(g * 16, 16)
                    src_v[i, sl] = src_v[i, sl] + off
                return cr
            lax.fori_loop(0, 16, offr, 0)

            def chunk(jj, cr):
                pltpu.async_copy(h_hbm.at[src_v.at[jj]], rows_a, sem_a).wait()
                _mul_rows(rows_a, src_v, dst_v, wts_v, ta_v, tb_v, jj)
                pltpu.sync_copy(rows_a, acc.at[dst_v.at[jj]], add=True)
                return cr
            lax.fori_loop(0, 16, chunk, 0)
            return carry

        lax.fori_loop(0, ncht // 16, outer, 0)

        plsc.subcore_barrier()
        for k in range(_RPS // _CHUNK):
            r0 = s * _RPS + k * _CHUNK
            pltpu.sync_copy(acc.at[pl.ds(r0, _CHUNK)],
                            out_hbm.at[c, pl.ds(r0, _CHUNK)])

    return pl.kernel(
        body,
        out_type=jax.ShapeDtypeStruct((2, N, _HC), jnp.float32),
        mesh=_SC_MESH,
        scratch_types=[
            pltpu.VMEM((16, _CHUNK), jnp.int32),
            pltpu.VMEM((16, _CHUNK), jnp.int32),
            pltpu.VMEM((16, _CHUNK), jnp.float32),
            pltpu.VMEM((N,), jnp.float32),
            pltpu.VMEM((2 * N,), jnp.float32),
            pltpu.VMEM((_CHUNK, _HC), jnp.float32),
            pltpu.VMEM_SHARED((N, _HC), jnp.float32),
            pltpu.SemaphoreType.DMA,
        ],
        compiler_params=pltpu.CompilerParams(
            use_tc_tiling_on_sc=False, needs_layout_passes=False),
    )


_WSEGSUM = _wsegsum_factory()


def _wsegsum(h, src2, dst2, wts2, ta, tb2):
    h2 = jnp.concatenate([h[:, :_HC], h[:, _HC:]], axis=0)
    p = _WSEGSUM(h2, src2, dst2, wts2, ta, tb2)
    return jnp.concatenate([p[0], p[1]], axis=1)


def _edge_softmax_factory():
    ncht = _NE // (_NTILES * _CHUNK)
    n3 = 3 * N

    def body(s_hbm, d_hbm, asad_hbm, ex_hbm, ss_hbm,
             src_v, dst_v, tb, hist, exb, sem):
        c = lax.axis_index("c")
        s = lax.axis_index("s")
        tid = s * 2 + c
        cbase = tid * ncht

        pltpu.sync_copy(s_hbm.at[pl.ds(cbase, ncht)], src_v)
        pltpu.sync_copy(d_hbm.at[pl.ds(cbase, ncht)], dst_v)
        pltpu.sync_copy(asad_hbm, tb)

        def zh(i, carry):
            z = jnp.zeros((16,), jnp.float32)
            for k in range(3):
                hist[k, pl.ds(i * 16, 16)] = z
            return carry
        lax.fori_loop(0, N // 16, zh, 0)

        iota = lax.iota(jnp.int32, 16)
        nreal = jnp.int32(E + N)

        def outer(ko, carry):
            for kc in range(8):
                j = ko * 8 + kc
                for g in range(8):
                    sl16 = pl.ds(g * 16, 16)
                    s16 = src_v[j, sl16]
                    d16 = dst_v[j, sl16]
                    gbase = (cbase + j) * _CHUNK + g * 16
                    msk = (gbase + iota) < nreal
                    for k in range(3):
                        av = plsc.load_gather(tb.at[k], [s16])
                        bv = plsc.load_gather(tb.at[3 + k], [d16])
                        ev = av + bv
                        ev = jnp.where(ev > 0, ev, 0.2 * ev)
                        xv = jnp.where(msk, jnp.exp(ev), 0.0)
                        plsc.addupdate_scatter(hist.at[k], [d16], xv)
                        exb[k, kc, sl16] = xv
            for k in range(3):
                pltpu.sync_copy(
                    exb.at[k],
                    ex_hbm.at[k, pl.ds(cbase + ko * 8, 8)])
            return carry

        lax.fori_loop(0, ncht // 8, outer, 0)

        pltpu.sync_copy(hist, ss_hbm.at[tid])

    return pl.kernel(
        body,
        out_type=(
            jax.ShapeDtypeStruct((3, _NE // _CHUNK, _CHUNK), jnp.float32),
            jax.ShapeDtypeStruct((_NTILES, 3, N), jnp.float32),
        ),
        mesh=_SC_MESH,
        scratch_types=[
            pltpu.VMEM((ncht, _CHUNK), jnp.int32),
            pltpu.VMEM((ncht, _CHUNK), jnp.int32),
            pltpu.VMEM((6, N), jnp.float32),
            pltpu.VMEM((3, N), jnp.float32),
            pltpu.VMEM((3, 8, _CHUNK), jnp.float32),
            pltpu.SemaphoreType.DMA,
        ],
        compiler_params=pltpu.CompilerParams(use_tc_tiling_on_sc=False, needs_layout_passes=False),
    )


def _deg_factory():
    ncht = _NE // (_NTILES * _CHUNK)

    def body(s_hbm, deg_hbm, src_v, hist, sem):
        c = lax.axis_index("c")
        s = lax.axis_index("s")
        tid = s * 2 + c
        cbase = tid * ncht

        pltpu.sync_copy(s_hbm.at[pl.ds(cbase, ncht)], src_v)

        def zh(i, carry):
            hist[pl.ds(i * 16, 16)] = jnp.zeros((16,), jnp.float32)
            return carry
        lax.fori_loop(0, N // 16, zh, 0)

        iota = lax.iota(jnp.int32, 16)
        nreal = jnp.int32(E)

        def outer(j, carry):
            for g in range(8):
                sl16 = pl.ds(g * 16, 16)
                s16 = src_v[j, sl16]
                gbase = (cbase + j) * _CHUNK + g * 16
                ones = jnp.where((gbase + iota) < nreal, 1.0, 0.0)
                plsc.addupdate_scatter(hist, [s16], ones)
            return carry

        lax.fori_loop(0, ncht, outer, 0)
        pltpu.sync_copy(hist, deg_hbm.at[tid])

    return pl.kernel(
        body,
        out_type=jax.ShapeDtypeStruct((_NTILES, N), jnp.float32),
        mesh=_SC_MESH,
        scratch_types=[
            pltpu.VMEM((ncht, _CHUNK), jnp.int32),
            pltpu.VMEM((N,), jnp.float32),
            pltpu.SemaphoreType.DMA,
        ],
        compiler_params=pltpu.CompilerParams(use_tc_tiling_on_sc=False, needs_layout_passes=False),
    )


_EDGE_SOFTMAX = _edge_softmax_factory()
_DEG = _deg_factory()


_BLK = 1024


def _mm_relu_kernel(x_ref, w_ref, b_ref, o_ref):
    o_ref[...] = jnp.maximum(
        jnp.dot(x_ref[...], w_ref[...], preferred_element_type=jnp.float32,
                  precision=lax.Precision.HIGHEST)
        + b_ref[...], 0.0)


def _emb(x, W, b):
    return pl.pallas_call(
        _mm_relu_kernel,
        grid=(N // _BLK,),
        in_specs=[
            pl.BlockSpec((_BLK, F_IN), lambda i: (i, 0)),
            pl.BlockSpec((F_IN, H), lambda i: (0, 0)),
            pl.BlockSpec((1, H), lambda i: (0, 0)),
        ],
        out_specs=pl.BlockSpec((_BLK, H), lambda i: (i, 0)),
        out_shape=jax.ShapeDtypeStruct((N, H), jnp.float32),
    )(x, W, b.reshape(1, H))


def _mm3_kernel(x0_ref, x1_ref, x2_ref, w_ref, b_ref, o_ref):
    acc = jnp.dot(x0_ref[...], w_ref[0], preferred_element_type=jnp.float32,
                  precision=lax.Precision.HIGHEST)
    acc += jnp.dot(x1_ref[...], w_ref[1], preferred_element_type=jnp.float32,
                  precision=lax.Precision.HIGHEST)
    acc += jnp.dot(x2_ref[...], w_ref[2], preferred_element_type=jnp.float32,
                  precision=lax.Precision.HIGHEST)
    o_ref[...] = acc + b_ref[...]


def _mm3(x0, x1, x2, W3, b):
    return pl.pallas_call(
        _mm3_kernel,
        grid=(N // _BLK,),
        in_specs=[
            pl.BlockSpec((_BLK, H), lambda i: (i, 0)),
            pl.BlockSpec((_BLK, H), lambda i: (i, 0)),
            pl.BlockSpec((_BLK, H), lambda i: (i, 0)),
            pl.BlockSpec((3, H, H), lambda i: (0, 0, 0)),
            pl.BlockSpec((1, H), lambda i: (0, 0)),
        ],
        out_specs=pl.BlockSpec((_BLK, H), lambda i: (i, 0)),
        out_shape=jax.ShapeDtypeStruct((N, H), jnp.float32),
    )(x0, x1, x2, W3, b.reshape(1, H))


def _stats_kernel(x_ref, cb_ref, o_ref):
    i = pl.program_id(0)
    x = x_ref[...] + cb_ref[...]
    s1 = jnp.sum(x, axis=0, keepdims=True)
    s2 = jnp.sum(x * x, axis=0, keepdims=True)
    st = jnp.concatenate([s1, s2], axis=0)

    @pl.when(i == 0)
    def _():
        o_ref[...] = st

    @pl.when(i > 0)
    def _():
        o_ref[...] = o_ref[...] + st


def _stats(x, cb):
    w = x.shape[1]
    return pl.pallas_call(
        _stats_kernel,
        grid=(N // _BLK,),
        in_specs=[
            pl.BlockSpec((_BLK, w), lambda i: (i, 0)),
            pl.BlockSpec((1, w), lambda i: (0, 0)),
        ],
        out_specs=pl.BlockSpec((2, w), lambda i: (0, 0)),
        out_shape=jax.ShapeDtypeStruct((2, w), jnp.float32),
    )(x, cb.reshape(1, w))


def _gatw_kernel(x_ref, c1_ref, c0_ref, w_ref, att_ref, hw_ref, as_ref):
    xb = x_ref[...] * c1_ref[...] + c0_ref[...]
    hw = jnp.dot(xb, w_ref[...], preferred_element_type=jnp.float32,
                  precision=lax.Precision.HIGHEST)
    hw_ref[...] = hw
    as_ref[...] = jnp.dot(hw, att_ref[...], preferred_element_type=jnp.float32,
                  precision=lax.Precision.HIGHEST)


def _gatw(x, c1, c0, W, attT):
    return pl.pallas_call(
        _gatw_kernel,
        grid=(N // _BLK,),
        in_specs=[
            pl.BlockSpec((_BLK, H), lambda i: (i, 0)),
            pl.BlockSpec((1, H), lambda i: (0, 0)),
            pl.BlockSpec((1, H), lambda i: (0, 0)),
            pl.BlockSpec((H, HEADS * H), lambda i: (0, 0)),
            pl.BlockSpec((HEADS * H, 8), lambda i: (0, 0)),
        ],
        out_specs=[
            pl.BlockSpec((_BLK, HEADS * H), lambda i: (i, 0)),
            pl.BlockSpec((_BLK, 8), lambda i: (i, 0)),
        ],
        out_shape=[
            jax.ShapeDtypeStruct((N, HEADS * H), jnp.float32),
            jax.ShapeDtypeStruct((N, 8), jnp.float32),
        ],
    )(x, c1.reshape(1, H), c0.reshape(1, H), W, attT)


def _linaff_kernel(x_ref, c1_ref, c0_ref, w_ref, b_ref, o_ref):
    xb = x_ref[...] * c1_ref[...] + c0_ref[...]
    o_ref[...] = (jnp.dot(xb, w_ref[...], preferred_element_type=jnp.float32,
                  precision=lax.Precision.HIGHEST)
                  + b_ref[...])


def _linaff(x, c1, c0, W, b):
    kin = x.shape[1]
    return pl.pallas_call(
        _linaff_kernel,
        grid=(N // _BLK,),
        in_specs=[
            pl.BlockSpec((_BLK, kin), lambda i: (i, 0)),
            pl.BlockSpec((1, kin), lambda i: (0, 0)),
            pl.BlockSpec((1, kin), lambda i: (0, 0)),
            pl.BlockSpec((kin, H), lambda i: (0, 0)),
            pl.BlockSpec((1, H), lambda i: (0, 0)),
        ],
        out_specs=pl.BlockSpec((_BLK, H), lambda i: (i, 0)),
        out_shape=jax.ShapeDtypeStruct((N, H), jnp.float32),
    )(x, c1.reshape(1, kin), c0.reshape(1, kin), W, b.reshape(1, H))


def _affine_relu_kernel(x_ref, c1_ref, c0_ref, o_ref):
    o_ref[...] = jnp.maximum(x_ref[...] * c1_ref[...] + c0_ref[...], 0.0)


def _affine_relu(x, c1, c0):
    return pl.pallas_call(
        _affine_relu_kernel,
        grid=(N // _BLK,),
        in_specs=[
            pl.BlockSpec((_BLK, H), lambda i: (i, 0)),
            pl.BlockSpec((1, H), lambda i: (0, 0)),
            pl.BlockSpec((1, H), lambda i: (0, 0)),
        ],
        out_specs=pl.BlockSpec((_BLK, H), lambda i: (i, 0)),
        out_shape=jax.ShapeDtypeStruct((N, H), jnp.float32),
    )(x, c1.reshape(1, H), c0.reshape(1, H))


def _rowscale_kernel(x_ref, r_ref, o_ref):
    o_ref[...] = x_ref[...] * r_ref[...]


def _rowscale(x, r):
    return pl.pallas_call(
        _rowscale_kernel,
        grid=(N // _BLK,),
        in_specs=[
            pl.BlockSpec((_BLK, H), lambda i: (i, 0)),
            pl.BlockSpec((_BLK, 1), lambda i: (i, 0)),
        ],
        out_specs=pl.BlockSpec((_BLK, H), lambda i: (i, 0)),
        out_shape=jax.ShapeDtypeStruct((N, H), jnp.float32),
    )(x, r.reshape(N, 1))


def _add2rs_kernel(p_ref, r_ref, o_ref):
    o_ref[...] = (p_ref[0] + p_ref[1]) * r_ref[...]


def _add2rs(p, r):
    return pl.pallas_call(
        _add2rs_kernel,
        grid=(N // _BLK,),
        in_specs=[
            pl.BlockSpec((2, _BLK, H), lambda i: (0, i, 0)),
            pl.BlockSpec((_BLK, 1), lambda i: (i, 0)),
        ],
        out_specs=pl.BlockSpec((_BLK, H), lambda i: (i, 0)),
        out_shape=jax.ShapeDtypeStruct((N, H), jnp.float32),
    )(p, r.reshape(N, 1))


def _add4_kernel(x_ref, c1_ref, c0_ref, a_ref, b_ref, d_ref, o_ref):
    o_ref[...] = (x_ref[...] * c1_ref[...] + c0_ref[...]
                  + a_ref[...] + b_ref[...] + d_ref[...])


def _add4(x, c1, c0, g0, g1, g2):
    return pl.pallas_call(
        _add4_kernel,
        grid=(N // _BLK,),
        in_specs=[
            pl.BlockSpec((_BLK, H), lambda i: (i, 0)),
            pl.BlockSpec((1, H), lambda i: (0, 0)),
            pl.BlockSpec((1, H), lambda i: (0, 0)),
            pl.BlockSpec((_BLK, H), lambda i: (i, 0)),
            pl.BlockSpec((_BLK, H), lambda i: (i, 0)),
            pl.BlockSpec((_BLK, H), lambda i: (i, 0)),
        ],
        out_specs=pl.BlockSpec((_BLK, H), lambda i: (i, 0)),
        out_shape=jax.ShapeDtypeStruct((N, H), jnp.float32),
    )(x, c1.reshape(1, H), c0.reshape(1, H), g0, g1, g2)


def _reduce32_kernel_rsqrt(x_ref, o_ref):
    s = jnp.sum(x_ref[...], axis=0, keepdims=True)
    o_ref[...] = jnp.where(s > 0, lax.rsqrt(jnp.maximum(s, 1e-30)), 0.0)


def _reduce32_kernel_recip(x_ref, o_ref):
    s = jnp.sum(x_ref[...], axis=0, keepdims=True)
    o_ref[...] = 1.0 / s


def _reduce32(x, post):
    x = x.reshape(_NTILES, -1)
    mm = x.shape[1]
    blk = 2048
    kern = (_reduce32_kernel_rsqrt if post == 'rsqrt'
            else _reduce32_kernel_recip)
    out = pl.pallas_call(
        kern,
        grid=(mm // blk,),
        in_specs=[pl.BlockSpec((_NTILES, blk), lambda i: (0, i))],
        out_specs=pl.BlockSpec((1, blk), lambda i: (0, i)),
        out_shape=jax.ShapeDtypeStruct((1, mm), jnp.float32),
    )(x)
    return out.reshape(mm)


def _bn_affine(x, cb, gamma, beta):
    mu = _stats(x, cb)[0] / N
    var = _stats(x, cb - mu)[1] / N
    c1 = gamma / jnp.sqrt(var + 1e-5)
    c0 = beta - mu * c1
    return c1, c0


def kernel(x, params, edge_index):
    src = edge_index[0]
    dst = edge_index[1]
    loop = jnp.arange(N, dtype=src.dtype)
    zc = jnp.zeros((_NE - E,), src.dtype)
    zg = jnp.zeros((_NE - (E + N),), src.dtype)
    src2 = jnp.concatenate([src, zc]).reshape(_NE // _CHUNK, _CHUNK)
    dst2 = jnp.concatenate([dst, zc]).reshape(_NE // _CHUNK, _CHUNK)
    s2 = jnp.concatenate([src, loop, zg]).reshape(_NE // _CHUNK, _CHUNK)
    d2 = jnp.concatenate([dst, loop, zg]).reshape(_NE // _CHUNK, _CHUNK)
    wts_cheb = jnp.concatenate(
        [jnp.full((E,), -1.0, jnp.float32),
         jnp.zeros((_NE - E,), jnp.float32)]
    ).reshape(src2.shape)

    dinv = _reduce32(_DEG(src2), 'rsqrt')
    dinv2 = jnp.concatenate([dinv, dinv])
    ones2 = jnp.ones((2 * N,), jnp.float32)
    p = params
    zH = jnp.zeros((H,), jnp.float32)

    def prop(h):
        return _wsegsum(h, src2, dst2, wts_cheb, dinv, dinv2)

    cur = _emb(x, p['W_emb'], p['b_emb'])
    gats = []
    for l in range(3):
        W3 = jnp.stack([p[f'cheb_W{l}'][0] - p[f'cheb_W{l}'][2],
                        p[f'cheb_W{l}'][1],
                        2.0 * p[f'cheb_W{l}'][2]])
        tx1 = prop(cur)
        p2 = prop(tx1)
        cheb_out = _mm3(cur, tx1, p2, W3, p[f'cheb_b{l}'])
        c1, c0 = _bn_affine(cheb_out, zH, p['bn_gamma'], p['bn_beta'])
        attT = jnp.zeros((HEADS * H, 8), jnp.float32)
        for k in range(HEADS):
            attT = attT.at[k * H:(k + 1) * H, k].set(p[f'gat{l}_att_src'][k])
            attT = attT.at[k * H:(k + 1) * H, 3 + k].set(
                p[f'gat{l}_att_dst'][k])
        hw, as8 = _gatw(cheb_out, c1, c0, p[f'gat{l}_W'], attT)
        asad = as8[:, :6].T
        ex, ssp = _EDGE_SOFTMAX(s2, d2, asad)
        rinv = _reduce32(ssp, 'recip').reshape(HEADS, N)
        aggs = [_wsegsum(hw[:, k * H:(k + 1) * H], s2, d2, ex[k],
                         rinv[k], ones2) for k in range(HEADS)]
        gat_cat = jnp.concatenate(aggs, axis=1)
        gc1, gc0 = _bn_affine(gat_cat, p[f'gat{l}_bias'],
                              p[f'bn{l}_gamma'], p[f'bn{l}_beta'])
        gc0 = gc0 + p[f'gat{l}_bias'] * gc1
        gats.append(_linaff(gat_cat, gc1, gc0,
                            p[f'lin{l}_W'], p[f'lin{l}_b']))
        if l < 2:
            cur = _affine_relu(cheb_out, c1, c0)
    return _add4(cheb_out, c1, c0, gats[0], gats[1], gats[2])
